# Initial kernel scaffold; baseline (speedup 1.0000x reference)
import jax
import jax.numpy as jnp
from jax.experimental import pallas as pl


def kernel(feats, coords, kernel):
    raise NotImplementedError("write your pallas kernel here")


# speedup vs baseline: 2.5545x; 2.5545x over previous
import functools

import jax
import jax.numpy as jnp
from jax import lax
from jax.experimental import pallas as pl
from jax.experimental.pallas import tpu as pltpu
from jax.experimental.pallas import tpu_sc as plsc

N = 50000
CIN = 64
COUT = 64
K = 32
S = 32
NCELL = S * S * S
HALF = NCELL // 2
CH = 512
NP = 50176
NCHUNK = NP // CH
DUMP = HALF
TAIL = NP - NCELL
TPT = TAIL // 32

_f32 = jnp.float32
_i32 = jnp.int32


def _s1_body(f_ref, c_ref, w_ref, of_ref, key_ref):
    f = f_ref[...]
    c = c_ref[...]
    kidx = c[:, 1:2]
    key_ref[...] = c[:, 0:1] * (S * S) + c[:, 2:3] * S + c[:, 3:4]
    acc = jnp.zeros((CH, COUT), _f32)
    for k in range(K):
        m = (kidx == k).astype(_f32)
        acc += jnp.dot(f, w_ref[k], preferred_element_type=_f32) * m
    of_ref[...] = acc


def _stage1(feats_p, coords_p, w):
    return pl.pallas_call(
        _s1_body,
        grid=(NCHUNK,),
        in_specs=[
            pl.BlockSpec((CH, CIN), lambda i: (i, 0)),
            pl.BlockSpec((CH, 4), lambda i: (i, 0)),
            pl.BlockSpec((K, CIN, COUT), lambda i: (0, 0, 0)),
        ],
        out_specs=[
            pl.BlockSpec((CH, COUT), lambda i: (i, 0)),
            pl.BlockSpec((CH, 1), lambda i: (i, 0)),
        ],
        out_shape=[
            jax.ShapeDtypeStruct((NP, COUT), _f32),
            jax.ShapeDtypeStruct((NP, 1), _i32),
        ],
    )(feats_p, coords_p, w)


def _s2_body(keys_hbm, feats_hbm, grid_out, occ_out,
             grid_sh, occ_sh, fv, kv, lv, zv, ov):
    cid = lax.axis_index("c")
    sid = lax.axis_index("s")
    lo = cid * HALF

    def _zrow(r, _):
        for g in range(CIN // 16):
            zv[r, pl.ds(g * 16, 16)] = jnp.zeros((16,), _f32)
        ov[r, pl.ds(0, 16)] = jnp.zeros((16,), _f32)
        return 0
    lax.fori_loop(0, 128, _zrow, 0)
    for j in range(8):
        pltpu.sync_copy(zv, grid_sh.at[pl.ds(sid * 1024 + j * 128, 128)])
        pltpu.sync_copy(ov, occ_sh.at[pl.ds(sid * 1024 + j * 128, 128)])

    @pl.when(sid == 0)
    def _():
        pltpu.sync_copy(zv.at[pl.ds(0, 8)], grid_sh.at[pl.ds(HALF, 8)])
        pltpu.sync_copy(ov.at[pl.ds(0, 8)], occ_sh.at[pl.ds(HALF, 8)])

    def _orow(r, _):
        ov[r, pl.ds(0, 16)] = jnp.ones((16,), _f32)
        return 0
    lax.fori_loop(0, 128, _orow, 0)

    plsc.subcore_barrier()

    def _chunk(i, _):
        ci = sid + 16 * i

        @pl.when(ci < NCHUNK)
        def _():
            pltpu.sync_copy(keys_hbm.at[pl.ds(ci * CH, CH)], kv)
            pltpu.sync_copy(feats_hbm.at[pl.ds(ci * CH, CH)], fv)
            for g in range(CH // 16):
                k16 = kv[pl.ds(g * 16, 16)]
                mine = (k16 >= lo) & (k16 < lo + HALF)
                l16 = jnp.where(mine, k16 - lo, DUMP)
                lv[g // 8, pl.ds((g % 8) * 16, 16)] = l16
            for j in range(CH // 128):
                idx = lv.at[j]
                pltpu.sync_copy(fv.at[pl.ds(j * 128, 128)],
                                grid_sh.at[idx], add=True)
                pltpu.sync_copy(ov, occ_sh.at[idx], add=True)
        return 0
    lax.fori_loop(0, 7, _chunk, 0)

    plsc.subcore_barrier()

    pltpu.sync_copy(grid_sh.at[pl.ds(sid * 1024, 1024)],
                    grid_out.at[pl.ds(lo + sid * 1024, 1024)])
    pltpu.sync_copy(occ_sh.at[pl.ds(sid * 1024, 1024)],
                    occ_out.at[pl.ds(lo + sid * 1024, 1024)])


def _stage2(keys, out_feats):
    kern = pl.kernel(
        _s2_body,
        out_type=(
            jax.ShapeDtypeStruct((NCELL, COUT), _f32),
            jax.ShapeDtypeStruct((NCELL, 16), _f32),
        ),
        mesh=plsc.VectorSubcoreMesh(core_axis_name="c", subcore_axis_name="s"),
        compiler_params=pltpu.CompilerParams(use_tc_tiling_on_sc=False),
        scratch_types=[
            pltpu.VMEM_SHARED((HALF + 8, COUT), _f32),
            pltpu.VMEM_SHARED((HALF + 8, 16), _f32),
            pltpu.VMEM((CH, COUT), _f32),
            pltpu.VMEM((CH,), _i32),
            pltpu.VMEM((4, 128), _i32),
            pltpu.VMEM((128, COUT), _f32),
            pltpu.VMEM((128, 16), _f32),
        ],
    )
    return kern(keys, out_feats)


def _s3_body(occv_ref, occ2k_ref, sidx_ref, irows_ref):
    m = (occv_ref[...] > 0.0).astype(_f32)
    su = (lax.broadcasted_iota(_i32, (128, 128), 0)
          < lax.broadcasted_iota(_i32, (128, 128), 1)).astype(_f32)
    inner = jnp.dot(m, su, preferred_element_type=_f32)
    rs = jnp.sum(m, axis=1, keepdims=True)
    lt = (lax.broadcasted_iota(_i32, (256, 256), 0)
          > lax.broadcasted_iota(_i32, (256, 256), 1)).astype(_f32)
    roff = jnp.dot(lt, rs, preferred_element_type=_f32)
    pocc = (inner + roff).astype(_i32)
    total = jnp.sum(m).astype(_i32)
    kg = (lax.broadcasted_iota(_i32, (256, 128), 0) * 128
          + lax.broadcasted_iota(_i32, (256, 128), 1))
    sidx_ref[...] = jnp.where(m > 0.0, pocc, total + kg - pocc)

    occ2k = occ2k_ref[...] > 0.0
    r = lax.broadcasted_iota(_i32, (2048, 256), 0)
    c = lax.broadcasted_iota(_i32, (2048, 256), 1)
    k2 = r * 16 + (c >> 4)
    lane = c & 15
    dec = jnp.where(lane == 0, k2 >> 10,
                    jnp.where(lane == 2, (k2 >> 5) & 31,
                              jnp.where(lane == 3, k2 & 31, 0)))
    fill = jnp.where(lane == 0, -1, 31)
    irows_ref[...] = jnp.where(occ2k, dec, fill)


def _stage3(occv, occ2k):
    return pl.pallas_call(
        _s3_body,
        out_shape=[
            jax.ShapeDtypeStruct((256, 128), _i32),
            jax.ShapeDtypeStruct((2048, 256), _i32),
        ],
    )(occv, occ2k)


def _s4_body(grid_hbm, irows_hbm, sidx_hbm, vals_out, idx_out,
             sidx, gv, iv, zv, tv):
    cid = lax.axis_index("c")
    sid = lax.axis_index("s")
    w = cid * 16 + sid
    base = w * 1024

    pltpu.sync_copy(sidx_hbm.at[pl.ds(w * 8, 8)], sidx)
    for j in range(8):
        pltpu.sync_copy(grid_hbm.at[pl.ds(base + j * 128, 128)], gv)
        pltpu.sync_copy(irows_hbm.at[pl.ds(base + j * 128, 128)], iv)
        pltpu.sync_copy(gv, vals_out.at[sidx.at[j]])
        pltpu.sync_copy(iv, idx_out.at[sidx.at[j]])

    lane = lax.broadcasted_iota(_i32, (16,), 0)
    fill16 = jnp.where(lane == 0, -1, 31)

    def _frow(r, _):
        for g in range(COUT // 16):
            zv[r, pl.ds(g * 16, 16)] = jnp.zeros((16,), _f32)
        tv[r, pl.ds(0, 16)] = fill16
        return 0
    lax.fori_loop(0, TPT, _frow, 0)
    pltpu.sync_copy(zv, vals_out.at[pl.ds(NCELL + w * TPT, TPT)])
    pltpu.sync_copy(tv, idx_out.at[pl.ds(NCELL + w * TPT, TPT)])


def _stage4(grid, irows, sidx):
    kern = pl.kernel(
        _s4_body,
        out_type=(
            jax.ShapeDtypeStruct((NP, COUT), _f32),
            jax.ShapeDtypeStruct((NP, 16), _i32),
        ),
        mesh=plsc.VectorSubcoreMesh(core_axis_name="c", subcore_axis_name="s"),
        compiler_params=pltpu.CompilerParams(use_tc_tiling_on_sc=False),
        scratch_types=[
            pltpu.VMEM((8, 128), _i32),
            pltpu.VMEM((128, COUT), _f32),
            pltpu.VMEM((128, 16), _i32),
            pltpu.VMEM((TPT, COUT), _f32),
            pltpu.VMEM((TPT, 16), _i32),
        ],
    )
    return kern(grid, irows, sidx)


@jax.jit
def kernel(feats, coords, kernel):
    feats_p = jnp.pad(feats, ((0, NP - N), (0, 0)))
    coords_p = jnp.pad(coords, ((0, NP - N), (0, 0)), constant_values=-1)

    out_feats, keys2d = _stage1(feats_p, coords_p, kernel)
    keys = keys2d.reshape(NP)

    grid, occ16 = _stage2(keys, out_feats)
    occv = occ16[:, 0].reshape(256, 128)

    sidx, irows2k = _stage3(occv, occ16.reshape(2048, 256))
    irows = irows2k.reshape(NCELL, 16)

    vals, idx16 = _stage4(grid, irows, sidx)
    return vals[:N], idx16[:N, :4]


# speedup vs baseline: 2.7636x; 1.0819x over previous
import functools

import jax
import jax.numpy as jnp
from jax import lax
from jax.experimental import pallas as pl
from jax.experimental.pallas import tpu as pltpu
from jax.experimental.pallas import tpu_sc as plsc

N = 50000
CIN = 64
COUT = 64
K = 32
S = 32
NCELL = S * S * S
HALF = NCELL // 2
CH = 512
NP = 50176
NCHUNK = NP // CH
DUMP = HALF
TAIL = NP - NCELL
TPT = TAIL // 32
VTPT = (N - NCELL) // 32
VREM = (N - NCELL) - 32 * VTPT

_f32 = jnp.float32
_i32 = jnp.int32


def _s1_body(f_ref, c_ref, wf_ref, of_ref, key_ref):
    f = f_ref[...]
    c = c_ref[...]
    kidx = c[:, 1:2]
    rows = (pl.program_id(0) * CH
            + lax.broadcasted_iota(_i32, (CH, 1), 0))
    key = c[:, 0:1] * (S * S) + c[:, 2:3] * S + c[:, 3:4]
    key_ref[...] = jnp.where(rows < N, key, -1)
    acc = jnp.zeros((CH, COUT), _f32)
    for k in range(K):
        m = (kidx == k).astype(_f32)
        acc += jnp.dot(f, wf_ref[k], preferred_element_type=_f32) * m
    of_ref[...] = acc


def _stage1(feats, coords, wf):
    return pl.pallas_call(
        _s1_body,
        grid=(NCHUNK,),
        in_specs=[
            pl.BlockSpec((CH, CIN), lambda i: (i, 0)),
            pl.BlockSpec((CH, 4), lambda i: (i, 0)),
            pl.BlockSpec((K, CIN, COUT), lambda i: (0, 0, 0)),
        ],
        out_specs=[
            pl.BlockSpec((CH, COUT), lambda i: (i, 0)),
            pl.BlockSpec((CH, 1), lambda i: (i, 0)),
        ],
        out_shape=[
            jax.ShapeDtypeStruct((NP, COUT), _f32),
            jax.ShapeDtypeStruct((NP, 1), _i32),
        ],
    )(feats, coords, wf)


def _s2_body(keys_hbm, feats_hbm, grid_out, occ_out,
             grid_sh, occ_sh, fv, kv, lv, zv, ov):
    cid = lax.axis_index("c")
    sid = lax.axis_index("s")
    lo = cid * HALF

    def _zrow(r, _):
        for g in range(CIN // 16):
            zv[r, pl.ds(g * 16, 16)] = jnp.zeros((16,), _f32)
        ov[r, pl.ds(0, 16)] = jnp.zeros((16,), _f32)
        return 0
    lax.fori_loop(0, 128, _zrow, 0)
    for j in range(8):
        pltpu.sync_copy(zv, grid_sh.at[pl.ds(sid * 1024 + j * 128, 128)])
        pltpu.sync_copy(ov, occ_sh.at[pl.ds(sid * 1024 + j * 128, 128)])

    @pl.when(sid == 0)
    def _():
        pltpu.sync_copy(zv.at[pl.ds(0, 8)], grid_sh.at[pl.ds(HALF, 8)])
        pltpu.sync_copy(ov.at[pl.ds(0, 8)], occ_sh.at[pl.ds(HALF, 8)])

    def _orow(r, _):
        ov[r, pl.ds(0, 16)] = jnp.ones((16,), _f32)
        return 0
    lax.fori_loop(0, 128, _orow, 0)

    plsc.subcore_barrier()

    def _chunk(i, _):
        ci = sid + 16 * i

        @pl.when(ci < NCHUNK)
        def _():
            pltpu.sync_copy(keys_hbm.at[pl.ds(ci * CH, CH)], kv)
            pltpu.sync_copy(feats_hbm.at[pl.ds(ci * CH, CH)], fv)
            for g in range(CH // 16):
                k16 = kv[pl.ds(g * 16, 16)]
                mine = (k16 >= lo) & (k16 < lo + HALF)
                l16 = jnp.where(mine, k16 - lo, DUMP)
                lv[g // 8, pl.ds((g % 8) * 16, 16)] = l16
            for j in range(CH // 128):
                idx = lv.at[j]
                pltpu.sync_copy(fv.at[pl.ds(j * 128, 128)],
                                grid_sh.at[idx], add=True)
                pltpu.sync_copy(ov, occ_sh.at[idx], add=True)
        return 0
    lax.fori_loop(0, 7, _chunk, 0)

    plsc.subcore_barrier()

    pltpu.sync_copy(grid_sh.at[pl.ds(sid * 1024, 1024)],
                    grid_out.at[pl.ds(lo + sid * 1024, 1024)])
    pltpu.sync_copy(occ_sh.at[pl.ds(sid * 1024, 1024)],
                    occ_out.at[pl.ds(lo + sid * 1024, 1024)])


def _stage2(keys, out_feats):
    kern = pl.kernel(
        _s2_body,
        out_type=(
            jax.ShapeDtypeStruct((NCELL, COUT), _f32),
            jax.ShapeDtypeStruct((NCELL, 16), _f32),
        ),
        mesh=plsc.VectorSubcoreMesh(core_axis_name="c", subcore_axis_name="s"),
        compiler_params=pltpu.CompilerParams(use_tc_tiling_on_sc=False),
        scratch_types=[
            pltpu.VMEM_SHARED((HALF + 8, COUT), _f32),
            pltpu.VMEM_SHARED((HALF + 8, 16), _f32),
            pltpu.VMEM((CH, COUT), _f32),
            pltpu.VMEM((CH,), _i32),
            pltpu.VMEM((4, 128), _i32),
            pltpu.VMEM((128, COUT), _f32),
            pltpu.VMEM((128, 16), _f32),
        ],
    )
    return kern(keys, out_feats)


def _s3_body(occv_ref, occ2k_ref, sidx_ref, irows_ref):
    m = (occv_ref[...] > 0.0).astype(_f32)
    su = (lax.broadcasted_iota(_i32, (128, 128), 0)
          < lax.broadcasted_iota(_i32, (128, 128), 1)).astype(_f32)
    inner = jnp.dot(m, su, preferred_element_type=_f32)
    rs = jnp.sum(m, axis=1, keepdims=True)
    lt = (lax.broadcasted_iota(_i32, (256, 256), 0)
          > lax.broadcasted_iota(_i32, (256, 256), 1)).astype(_f32)
    roff = jnp.dot(lt, rs, preferred_element_type=_f32)
    pocc = (inner + roff).astype(_i32)
    total = jnp.sum(m).astype(_i32)
    kg = (lax.broadcasted_iota(_i32, (256, 128), 0) * 128
          + lax.broadcasted_iota(_i32, (256, 128), 1))
    sidx_ref[...] = jnp.where(m > 0.0, pocc, total + kg - pocc)

    occ2k = occ2k_ref[...] > 0.0
    r = lax.broadcasted_iota(_i32, (2048, 256), 0)
    c = lax.broadcasted_iota(_i32, (2048, 256), 1)
    k2 = r * 16 + (c >> 4)
    lane = c & 15
    dec = jnp.where(lane == 0, k2 >> 10,
                    jnp.where(lane == 2, (k2 >> 5) & 31,
                              jnp.where(lane == 3, k2 & 31, 0)))
    fill = jnp.where(lane == 0, -1, 31)
    irows_ref[...] = jnp.where(occ2k, dec, fill)


def _stage3(occv, occ2k):
    return pl.pallas_call(
        _s3_body,
        out_shape=[
            jax.ShapeDtypeStruct((256, 128), _i32),
            jax.ShapeDtypeStruct((2048, 256), _i32),
        ],
    )(occv, occ2k)


def _s4_body(grid_hbm, irows_hbm, sidx_hbm, vals_out, idx_out,
             sidx, gv, iv, zv, tv):
    cid = lax.axis_index("c")
    sid = lax.axis_index("s")
    w = cid * 16 + sid
    base = w * 1024

    pltpu.sync_copy(sidx_hbm.at[pl.ds(w * 8, 8)], sidx)
    for j in range(8):
        pltpu.sync_copy(grid_hbm.at[pl.ds(base + j * 128, 128)], gv)
        pltpu.sync_copy(irows_hbm.at[pl.ds(base + j * 128, 128)], iv)
        pltpu.sync_copy(gv, vals_out.at[sidx.at[j]])
        pltpu.sync_copy(iv, idx_out.at[sidx.at[j]])

    lane = lax.broadcasted_iota(_i32, (16,), 0)
    fill16 = jnp.where(lane == 0, -1, 31)

    def _frow(r, _):
        for g in range(COUT // 16):
            zv[r, pl.ds(g * 16, 16)] = jnp.zeros((16,), _f32)
        tv[r, pl.ds(0, 16)] = fill16
        return 0
    lax.fori_loop(0, TPT, _frow, 0)
    pltpu.sync_copy(zv.at[pl.ds(0, VTPT)],
                    vals_out.at[pl.ds(NCELL + w * VTPT, VTPT)])
    pltpu.sync_copy(tv, idx_out.at[pl.ds(NCELL + w * TPT, TPT)])

    @pl.when(w == 0)
    def _():
        pltpu.sync_copy(zv.at[pl.ds(0, VREM)],
                        vals_out.at[pl.ds(NCELL + 32 * VTPT, VREM)])


def _stage4(grid, irows, sidx):
    kern = pl.kernel(
        _s4_body,
        out_type=(
            jax.ShapeDtypeStruct((N, COUT), _f32),
            jax.ShapeDtypeStruct((NP, 16), _i32),
        ),
        mesh=plsc.VectorSubcoreMesh(core_axis_name="c", subcore_axis_name="s"),
        compiler_params=pltpu.CompilerParams(use_tc_tiling_on_sc=False),
        scratch_types=[
            pltpu.VMEM((8, 128), _i32),
            pltpu.VMEM((128, COUT), _f32),
            pltpu.VMEM((128, 16), _i32),
            pltpu.VMEM((TPT, COUT), _f32),
            pltpu.VMEM((TPT, 16), _i32),
        ],
    )
    return kern(grid, irows, sidx)


@jax.jit
def kernel(feats, coords, kernel):
    out_feats, keys2d = _stage1(feats, coords, kernel)
    keys = keys2d.reshape(NP)

    grid, occ16 = _stage2(keys, out_feats)
    occv = occ16[:, 0].reshape(256, 128)

    sidx, irows2k = _stage3(occv, occ16.reshape(2048, 256))
    irows = irows2k.reshape(NCELL, 16)

    vals, idx16 = _stage4(grid, irows, sidx)
    return vals, idx16[:N, :4]
